# Initial kernel scaffold; baseline (speedup 1.0000x reference)
import jax
import jax.numpy as jnp
from jax.experimental import pallas as pl


def kernel(node_type_index, node_sub_token_ids, type_table, token_table):
    raise NotImplementedError("write your pallas kernel here")


# speedup vs baseline: 1.7157x; 1.7157x over previous
import functools

import jax
import jax.numpy as jnp
from jax import lax
from jax.experimental import pallas as pl
from jax.experimental.pallas import tpu as pltpu
from jax.experimental.pallas import tpu_sc as plsc

_N = 16384
_L = 20
_D = 32
_HALF = 16

_NC = 2
_NS = 16
_NW = _NC * _NS
_NODES_PER_W = _N // _NW
_C = 128
_CHUNKS = _NODES_PER_W // _C
_IDX_ROWS = _C * _L // 128


def _sc_body(type_idx_hbm, sub_ids_hbm, type_table_hbm, token_table_hbm,
             out_hbm, idx_typ_v, idx_tok_v, tok_rows_v, typ_rows_v,
             out_v, sem):
    wid = lax.axis_index("s") * _NC + lax.axis_index("c")

    pltpu.sync_copy(type_idx_hbm.at[pl.ds(wid * _NODES_PER_W, _NODES_PER_W)],
                    idx_typ_v)
    pltpu.sync_copy(
        sub_ids_hbm.at[pl.ds(wid * (_NODES_PER_W * _L // 128),
                             _NODES_PER_W * _L // 128)],
        idx_tok_v)

    for c in range(_CHUNKS):
        nbase = wid * _NODES_PER_W + c * _C

        descs = [
            pltpu.async_copy(token_table_hbm.at[idx_tok_v.at[c * _IDX_ROWS + j]],
                             tok_rows_v.at[pl.ds(j * 128, 128)], sem)
            for j in range(_IDX_ROWS)
        ]
        descs.append(
            pltpu.async_copy(type_table_hbm.at[idx_typ_v.at[pl.ds(c * _C, _C)]],
                             typ_rows_v, sem))
        for dsc in descs:
            dsc.wait()

        def node_body(n, carry):
            base = n * _L
            lo = tok_rows_v[base, pl.ds(0, _HALF)]
            hi = tok_rows_v[base, pl.ds(_HALF, _HALF)]
            for l in range(1, _L):
                lo = lo + tok_rows_v[base + l, pl.ds(0, _HALF)]
                hi = hi + tok_rows_v[base + l, pl.ds(_HALF, _HALF)]
            out_v[n, pl.ds(0, _HALF)] = typ_rows_v[n, pl.ds(0, _HALF)]
            out_v[n, pl.ds(_HALF, _HALF)] = typ_rows_v[n, pl.ds(_HALF, _HALF)]
            out_v[n, pl.ds(2 * _HALF, _HALF)] = lo / float(_L)
            out_v[n, pl.ds(3 * _HALF, _HALF)] = hi / float(_L)
            return carry

        lax.fori_loop(0, _C, node_body, 0)

        pltpu.sync_copy(out_v, out_hbm.at[pl.ds(nbase, _C)])


def kernel(node_type_index, node_sub_token_ids, type_table, token_table):
    sub_ids_2d = node_sub_token_ids.reshape(_N * _L // 128, 128)

    mesh = plsc.VectorSubcoreMesh(core_axis_name="c", subcore_axis_name="s")
    run = pl.kernel(
        _sc_body,
        mesh=mesh,
        compiler_params=pltpu.CompilerParams(use_tc_tiling_on_sc=False),
        out_type=jax.ShapeDtypeStruct((_N, 2 * _D), jnp.float32),
        scratch_types=[
            pltpu.VMEM((_NODES_PER_W,), jnp.int32),
            pltpu.VMEM((_NODES_PER_W * _L // 128, 128), jnp.int32),
            pltpu.VMEM((_C * _L, _D), jnp.float32),
            pltpu.VMEM((_C, _D), jnp.float32),
            pltpu.VMEM((_C, 2 * _D), jnp.float32),
            pltpu.SemaphoreType.DMA,
        ],
    )
    return run(node_type_index, sub_ids_2d, type_table, token_table)


# speedup vs baseline: 1.7413x; 1.0149x over previous
import jax
import jax.numpy as jnp
from jax import lax
from jax.experimental import pallas as pl
from jax.experimental.pallas import tpu as pltpu
from jax.experimental.pallas import tpu_sc as plsc

_N = 16384
_L = 20
_D = 32
_HALF = 16

_NC = 2
_NS = 16
_NW = _NC * _NS
_NODES_PER_W = _N // _NW
_C = 64
_CHUNKS = _NODES_PER_W // _C
_IDX_ROWS = _C * _L // 128
_W_IDX_ROWS = _NODES_PER_W * _L // 128


def _sc_body(type_idx_hbm, sub_ids_hbm, type_table_hbm, token_table_hbm,
             out_hbm, idx_typ_v, idx_tok_v,
             tok_a, tok_b, typ_a, typ_b, out_a, out_b,
             sem_a, sem_b):
    wid = lax.axis_index("s") * _NC + lax.axis_index("c")

    tok_bufs = (tok_a, tok_b)
    typ_bufs = (typ_a, typ_b)
    out_bufs = (out_a, out_b)
    sems = (sem_a, sem_b)

    pltpu.sync_copy(type_idx_hbm.at[pl.ds(wid * _NODES_PER_W, _NODES_PER_W)],
                    idx_typ_v)
    pltpu.sync_copy(sub_ids_hbm.at[pl.ds(wid * _W_IDX_ROWS, _W_IDX_ROWS)],
                    idx_tok_v)

    def fire(c):
        b = c % 2
        descs = [
            pltpu.async_copy(token_table_hbm.at[idx_tok_v.at[c * _IDX_ROWS + j]],
                             tok_bufs[b].at[pl.ds(j * 128, 128)], sems[b])
            for j in range(_IDX_ROWS)
        ]
        descs.append(
            pltpu.async_copy(type_table_hbm.at[idx_typ_v.at[pl.ds(c * _C, _C)]],
                             typ_bufs[b], sems[b]))
        return descs

    descs = fire(0)
    for c in range(_CHUNKS):
        b = c % 2
        next_descs = fire(c + 1) if c + 1 < _CHUNKS else []
        for dsc in descs:
            dsc.wait()
        descs = next_descs

        tok_v = tok_bufs[b]
        typ_v = typ_bufs[b]
        out_v = out_bufs[b]

        @plsc.parallel_loop(0, _C, unroll=2)
        def node_body(n):
            base = n * _L
            lo = tok_v[base, pl.ds(0, _HALF)]
            hi = tok_v[base, pl.ds(_HALF, _HALF)]
            for l in range(1, _L):
                lo = lo + tok_v[base + l, pl.ds(0, _HALF)]
                hi = hi + tok_v[base + l, pl.ds(_HALF, _HALF)]
            out_v[n, pl.ds(0, _HALF)] = typ_v[n, pl.ds(0, _HALF)]
            out_v[n, pl.ds(_HALF, _HALF)] = typ_v[n, pl.ds(_HALF, _HALF)]
            out_v[n, pl.ds(2 * _HALF, _HALF)] = lo / float(_L)
            out_v[n, pl.ds(3 * _HALF, _HALF)] = hi / float(_L)

        nbase = wid * _NODES_PER_W + c * _C
        pltpu.sync_copy(out_v, out_hbm.at[pl.ds(nbase, _C)])


def kernel(node_type_index, node_sub_token_ids, type_table, token_table):
    sub_ids_2d = node_sub_token_ids.reshape(_N * _L // 128, 128)

    mesh = plsc.VectorSubcoreMesh(core_axis_name="c", subcore_axis_name="s")
    run = pl.kernel(
        _sc_body,
        mesh=mesh,
        compiler_params=pltpu.CompilerParams(use_tc_tiling_on_sc=False),
        out_type=jax.ShapeDtypeStruct((_N, 2 * _D), jnp.float32),
        scratch_types=[
            pltpu.VMEM((_NODES_PER_W,), jnp.int32),
            pltpu.VMEM((_W_IDX_ROWS, 128), jnp.int32),
            pltpu.VMEM((_C * _L, _D), jnp.float32),
            pltpu.VMEM((_C * _L, _D), jnp.float32),
            pltpu.VMEM((_C, _D), jnp.float32),
            pltpu.VMEM((_C, _D), jnp.float32),
            pltpu.VMEM((_C, 2 * _D), jnp.float32),
            pltpu.VMEM((_C, 2 * _D), jnp.float32),
            pltpu.SemaphoreType.DMA,
            pltpu.SemaphoreType.DMA,
        ],
    )
    return run(node_type_index, sub_ids_2d, type_table, token_table)
